# baseline (speedup 1.0000x reference)
import functools

import jax
import jax.numpy as jnp
from jax import lax
from jax.experimental import pallas as pl
from jax.experimental.pallas import tpu as pltpu
from jax.experimental.pallas import tpu_sc as plsc

_NUM_CORES = 2
_NUM_SUBCORES = 16
_NW = _NUM_CORES * _NUM_SUBCORES
_L = 16


def _sc_body(nframes, natom, ntypes, cols,
             coord_hbm, spin_hbm, atype_hbm, force_hbm, vsm_hbm,
             cs_hbm, as_hbm, fm_hbm, mk_hbm,
             coord_v, spin_v, atype_v, fmag_v,
             table_v, vmask_v, atspin_v, mask_v,
             sem_a, sem_c, sem_s, sem_fm, sem_o):
  wid = lax.axis_index("c") * _NUM_SUBCORES + lax.axis_index("s")
  c0 = wid * cols
  half = cols // 2
  g2 = half // _L

  in_a, in_c, in_s, in_fm = [], [], [], []
  for ch in range(2):
    hb = ch * half
    in_a.append(pltpu.async_copy(
        atype_hbm.at[:, pl.ds(c0 + hb, half)],
        atype_v.at[:, pl.ds(hb, half)], sem_a[ch]))
  for ch in range(2):
    hb = ch * half
    in_c.append(pltpu.async_copy(
        coord_hbm.at[:, :, pl.ds(c0 + hb, half)],
        coord_v.at[:, :, pl.ds(hb, half)], sem_c[ch]))
    in_s.append(pltpu.async_copy(
        spin_hbm.at[:, :, pl.ds(c0 + hb, half)],
        spin_v.at[:, :, pl.ds(hb, half)], sem_s[ch]))
    in_fm.append(pltpu.async_copy(
        force_hbm.at[:, :, pl.ds(natom + c0 + hb, half)],
        fmag_v.at[:, :, pl.ds(hb, half)], sem_fm[ch]))
  pltpu.sync_copy(vsm_hbm, table_v)

  outs = []
  for ch in range(2):
    hb = ch * half
    in_a[ch].wait()

    def body_atype(g, carry, hb=hb):
      r = g // g2
      cc = hb + (g % g2) * _L
      at = atype_v[r, pl.ds(cc, _L)]
      vm = plsc.load_gather(table_v, [at])
      vmask_v[r, pl.ds(cc, _L)] = vm
      atspin_v[r, pl.ds(cc, _L)] = at + ntypes
      mask_v[r, pl.ds(cc, _L)] = jnp.where(
          vm > 0.0, jnp.int32(1), jnp.int32(0))
      return carry

    lax.fori_loop(0, nframes * g2, body_atype, 0)

    outs.append(pltpu.async_copy(
        atype_v.at[:, pl.ds(hb, half)],
        as_hbm.at[:, pl.ds(c0 + hb, half)], sem_o))
    outs.append(pltpu.async_copy(
        atspin_v.at[:, pl.ds(hb, half)],
        as_hbm.at[:, pl.ds(natom + c0 + hb, half)], sem_o))
    outs.append(pltpu.async_copy(
        mask_v.at[:, pl.ds(hb, half)],
        mk_hbm.at[:, pl.ds(c0 + hb, half)], sem_o))

  for ch in range(2):
    hb = ch * half
    in_c[ch].wait()
    outs.append(pltpu.async_copy(
        coord_v.at[:, :, pl.ds(hb, half)],
        cs_hbm.at[:, :, pl.ds(c0 + hb, half)], sem_o))
    in_s[ch].wait()
    in_fm[ch].wait()

    def body_vec(g, carry, hb=hb):
      r = g // g2
      cc = hb + (g % g2) * _L
      vm = vmask_v[r, pl.ds(cc, _L)]
      for p in range(3):
        spin_v[p, r, pl.ds(cc, _L)] = (
            coord_v[p, r, pl.ds(cc, _L)] + spin_v[p, r, pl.ds(cc, _L)] * vm)
        fmag_v[p, r, pl.ds(cc, _L)] = fmag_v[p, r, pl.ds(cc, _L)] * vm
      return carry

    lax.fori_loop(0, nframes * g2, body_vec, 0)

    outs.append(pltpu.async_copy(
        spin_v.at[:, :, pl.ds(hb, half)],
        cs_hbm.at[:, :, pl.ds(natom + c0 + hb, half)], sem_o))
    outs.append(pltpu.async_copy(
        fmag_v.at[:, :, pl.ds(hb, half)],
        fm_hbm.at[:, :, pl.ds(c0 + hb, half)], sem_o))

  for cp in outs:
    cp.wait()


def kernel(coord, atype, spin, force, virtual_scale_mask):
  nframes, natom = coord.shape[0], coord.shape[1]
  ntypes = virtual_scale_mask.shape[0]
  assert natom % _NW == 0
  cols = natom // _NW
  assert cols % _L == 0

  mesh = plsc.VectorSubcoreMesh(
      core_axis_name="c", subcore_axis_name="s",
      num_cores=_NUM_CORES, num_subcores=_NUM_SUBCORES)

  f32, i32 = jnp.float32, jnp.int32
  run = pl.kernel(
      functools.partial(_sc_body, nframes, natom, ntypes, cols),
      out_type=[
          jax.ShapeDtypeStruct((3, nframes, 2 * natom), f32),
          jax.ShapeDtypeStruct((nframes, 2 * natom), i32),
          jax.ShapeDtypeStruct((3, nframes, natom), f32),
          jax.ShapeDtypeStruct((nframes, natom), i32),
      ],
      mesh=mesh,
      compiler_params=pltpu.CompilerParams(needs_layout_passes=False),
      scratch_types=[
          pltpu.VMEM((3, nframes, cols), f32),
          pltpu.VMEM((3, nframes, cols), f32),
          pltpu.VMEM((nframes, cols), i32),
          pltpu.VMEM((3, nframes, cols), f32),
          pltpu.VMEM((ntypes,), f32),
          pltpu.VMEM((nframes, cols), f32),
          pltpu.VMEM((nframes, cols), i32),
          pltpu.VMEM((nframes, cols), i32),
          [pltpu.SemaphoreType.DMA] * 2,
          [pltpu.SemaphoreType.DMA] * 2,
          [pltpu.SemaphoreType.DMA] * 2,
          [pltpu.SemaphoreType.DMA] * 2,
          pltpu.SemaphoreType.DMA,
      ],
  )

  cs_t, ats, fm_t, mk = run(
      jnp.transpose(coord, (2, 0, 1)), jnp.transpose(spin, (2, 0, 1)),
      atype, jnp.transpose(force, (2, 0, 1)), virtual_scale_mask)

  coord_spin = jnp.transpose(cs_t, (1, 2, 0))
  force_real = force[:, :natom]
  force_mag = jnp.transpose(fm_t, (1, 2, 0))
  atomic_mask = mk.reshape(nframes, natom, 1).astype(jnp.bool_)
  return coord_spin, ats, force_real, force_mag, atomic_mask


# speedup vs baseline: 1.0134x; 1.0134x over previous
import functools

import jax
import jax.numpy as jnp
from jax import lax
from jax.experimental import pallas as pl
from jax.experimental.pallas import tpu as pltpu
from jax.experimental.pallas import tpu_sc as plsc

_NUM_CORES = 2
_NUM_SUBCORES = 16
_NW = _NUM_CORES * _NUM_SUBCORES
_L = 16


def _sc_body(nframes, natom, ntypes, cols,
             coord_hbm, spin_hbm, atype_hbm, force_hbm, vsm_hbm,
             cs_hbm, as_hbm, fm_hbm, mk_hbm,
             coord_v, spin_v, atype_v, fmag_v,
             table_v, vmask_v, atspin_v, mask_v,
             sem_a, sem_c, sem_s, sem_fm, sem_o):
  wid = lax.axis_index("c") * _NUM_SUBCORES + lax.axis_index("s")
  c0 = wid * cols
  groups = cols // _L

  in_a = pltpu.async_copy(atype_hbm.at[:, pl.ds(c0, cols)], atype_v, sem_a)
  in_c = pltpu.async_copy(coord_hbm.at[:, :, pl.ds(c0, cols)], coord_v, sem_c)
  in_s = pltpu.async_copy(spin_hbm.at[:, :, pl.ds(c0, cols)], spin_v, sem_s)
  in_fm = pltpu.async_copy(
      force_hbm.at[:, :, pl.ds(natom + c0, cols)], fmag_v, sem_fm)
  pltpu.sync_copy(vsm_hbm, table_v)

  in_a.wait()

  def body_atype(g, carry):
    r = g // groups
    cc = (g % groups) * _L
    at = atype_v[r, pl.ds(cc, _L)]
    vm = plsc.load_gather(table_v, [at])
    vmask_v[r, pl.ds(cc, _L)] = vm
    atspin_v[r, pl.ds(cc, _L)] = at + ntypes
    mask_v[r, pl.ds(cc, _L)] = jnp.where(vm > 0.0, jnp.int32(1), jnp.int32(0))
    return carry

  lax.fori_loop(0, nframes * groups, body_atype, 0)

  out_ar = pltpu.async_copy(atype_v, as_hbm.at[:, pl.ds(c0, cols)], sem_o)
  out_av = pltpu.async_copy(
      atspin_v, as_hbm.at[:, pl.ds(natom + c0, cols)], sem_o)
  out_mk = pltpu.async_copy(mask_v, mk_hbm.at[:, pl.ds(c0, cols)], sem_o)

  in_c.wait()
  out_cr = pltpu.async_copy(coord_v, cs_hbm.at[:, :, pl.ds(c0, cols)], sem_o)
  in_s.wait()
  in_fm.wait()

  def body_vec(g, carry):
    r = g // groups
    cc = (g % groups) * _L
    vm = vmask_v[r, pl.ds(cc, _L)]
    for p in range(3):
      spin_v[p, r, pl.ds(cc, _L)] = (
          coord_v[p, r, pl.ds(cc, _L)] + spin_v[p, r, pl.ds(cc, _L)] * vm)
      fmag_v[p, r, pl.ds(cc, _L)] = fmag_v[p, r, pl.ds(cc, _L)] * vm
    return carry

  lax.fori_loop(0, nframes * groups, body_vec, 0)
  out_cv = pltpu.async_copy(
      spin_v, cs_hbm.at[:, :, pl.ds(natom + c0, cols)], sem_o)
  out_fm = pltpu.async_copy(fmag_v, fm_hbm.at[:, :, pl.ds(c0, cols)], sem_o)

  out_ar.wait()
  out_av.wait()
  out_mk.wait()
  out_cr.wait()
  out_cv.wait()
  out_fm.wait()


def kernel(coord, atype, spin, force, virtual_scale_mask):
  nframes, natom = coord.shape[0], coord.shape[1]
  ntypes = virtual_scale_mask.shape[0]
  assert natom % _NW == 0
  cols = natom // _NW
  assert cols % _L == 0

  mesh = plsc.VectorSubcoreMesh(
      core_axis_name="c", subcore_axis_name="s",
      num_cores=_NUM_CORES, num_subcores=_NUM_SUBCORES)

  f32, i32 = jnp.float32, jnp.int32
  run = pl.kernel(
      functools.partial(_sc_body, nframes, natom, ntypes, cols),
      out_type=[
          jax.ShapeDtypeStruct((3, nframes, 2 * natom), f32),
          jax.ShapeDtypeStruct((nframes, 2 * natom), i32),
          jax.ShapeDtypeStruct((3, nframes, natom), f32),
          jax.ShapeDtypeStruct((nframes, natom), i32),
      ],
      mesh=mesh,
      compiler_params=pltpu.CompilerParams(needs_layout_passes=False),
      scratch_types=[
          pltpu.VMEM((3, nframes, cols), f32),
          pltpu.VMEM((3, nframes, cols), f32),
          pltpu.VMEM((nframes, cols), i32),
          pltpu.VMEM((3, nframes, cols), f32),
          pltpu.VMEM((ntypes,), f32),
          pltpu.VMEM((nframes, cols), f32),
          pltpu.VMEM((nframes, cols), i32),
          pltpu.VMEM((nframes, cols), i32),
          pltpu.SemaphoreType.DMA,
          pltpu.SemaphoreType.DMA,
          pltpu.SemaphoreType.DMA,
          pltpu.SemaphoreType.DMA,
          pltpu.SemaphoreType.DMA,
      ],
  )

  cs_t, ats, fm_t, mk = run(
      jnp.transpose(coord, (2, 0, 1)), jnp.transpose(spin, (2, 0, 1)),
      atype, jnp.transpose(force, (2, 0, 1)), virtual_scale_mask)

  coord_spin = jnp.transpose(cs_t, (1, 2, 0))
  force_real = force[:, :natom]
  force_mag = jnp.transpose(fm_t, (1, 2, 0))
  atomic_mask = mk.reshape(nframes, natom, 1).astype(jnp.bool_)
  return coord_spin, ats, force_real, force_mag, atomic_mask
